# baseline (speedup 1.0000x reference)
import functools

import jax
import jax.numpy as jnp
from jax.experimental import pallas as pl

EPS = 1e-8


def _attn_kernel(q_ref, k_ref, v_ref, rot_ref, o_ref, *, scale, n_hashes):
    q = q_ref[0]
    k = k_ref[0]
    v = v_ref[0]
    rot = rot_ref[0]

    qn = q / (jnp.sqrt(jnp.sum(q * q, axis=-1, keepdims=True)) + 1e-8)
    kn = k / (jnp.sqrt(jnp.sum(k * k, axis=-1, keepdims=True)) + 1e-8)
    qbits = jnp.dot(qn, rot.T, preferred_element_type=jnp.float32) > 0
    kbits = jnp.dot(kn, rot.T, preferred_element_type=jnp.float32) > 0
    powers = 2 ** jnp.arange(n_hashes, dtype=jnp.int32)
    qcode = jnp.sum(qbits.astype(jnp.int32) * powers, axis=-1)
    kcode = jnp.sum(kbits.astype(jnp.int32) * powers, axis=-1)

    mask = qcode[:, None] == kcode[None, :]

    scores = jnp.dot(q, k.T, preferred_element_type=jnp.float32) * scale
    neg = jnp.finfo(jnp.float32).min
    scores_m = jnp.where(mask, scores, neg)
    m = jnp.max(scores_m, axis=-1, keepdims=True)
    p = jnp.where(mask, jnp.exp(scores_m - m), 0.0)
    l = jnp.sum(p, axis=-1, keepdims=True)
    acc = jnp.dot(p, v, preferred_element_type=jnp.float32)
    o_ref[0] = acc / ((l + EPS) * n_hashes)


@functools.partial(jax.jit, static_argnames=())
def kernel(Q, K, V, rotations):
    B, H, S, D = Q.shape
    NH = rotations.shape[1]
    TQ = 256
    scale = 1.0 / (D ** 0.5)

    q = Q.reshape(B * H, S, D)
    k = K.reshape(B * H, S, D)
    v = V.reshape(B * H, S, D)
    rot = jnp.broadcast_to(rotations[None], (B, H, NH, D)).reshape(B * H, NH, D)

    grid = (B * H, S // TQ)
    out = pl.pallas_call(
        functools.partial(_attn_kernel, scale=scale, n_hashes=NH),
        grid=grid,
        in_specs=[
            pl.BlockSpec((1, TQ, D), lambda h, i: (h, i, 0)),
            pl.BlockSpec((1, S, D), lambda h, i: (h, 0, 0)),
            pl.BlockSpec((1, S, D), lambda h, i: (h, 0, 0)),
            pl.BlockSpec((1, NH, D), lambda h, i: (h, 0, 0)),
        ],
        out_specs=pl.BlockSpec((1, TQ, D), lambda h, i: (h, i, 0)),
        out_shape=jax.ShapeDtypeStruct((B * H, S, D), jnp.float32),
    )(q, k, v, rot)
    return out.reshape(B, H, S, D)
